# baseline (speedup 1.0000x reference)
import dataclasses
import functools

import jax
import jax.numpy as jnp
from jax import lax
from jax.experimental import pallas as pl
from jax.experimental.pallas import tpu as pltpu
from jax.experimental.pallas import tpu_sc as plsc

GS = 128
NLEV = 16
L = 16
BLOCK = 16384
GPI = 2
INV_MAXQ2 = 2.0 / 15.0


def kernel(x, lookup_values):
    shape = x.shape
    n = x.size
    x1 = x.reshape(n)
    mesh = plsc.VectorSubcoreMesh(core_axis_name="c", subcore_axis_name="s")
    cp = pltpu.CompilerParams()
    if "needs_layout_passes" in pltpu.CompilerParams.__dataclass_fields__:
        cp = dataclasses.replace(cp, needs_layout_passes=False)

    @functools.partial(
        pl.kernel,
        mesh=mesh,
        out_type=jax.ShapeDtypeStruct((n,), jnp.float32),
        scratch_types=[pltpu.VMEM((NLEV,), jnp.float32)],
        compiler_params=cp,
    )
    def run(x_hbm, lut_hbm, o_hbm, lut_vmem):
        pltpu.sync_copy(lut_hbm, lut_vmem)
        lut = lut_vmem[...]
        cbv = [jnp.full((L,), lut[i], jnp.float32) for i in range(NLEV)]
        midv = [(cbv[i] + cbv[i + 1]) * 0.5 for i in range(NLEV - 1)]

        def nearest(q):
            r = cbv[0]
            for k in range(1, NLEV):
                r = jnp.where(q > midv[k - 1], cbv[k], r)
            return r

        zv = nearest(jnp.full((L,), 8.0, jnp.float32))

        def body(x_vmem, o_vmem):
            @pl.loop(0, BLOCK // GS, step=GPI)
            def _(g0):
                for gg in range(GPI):
                    base = (g0 + gg) * GS
                    xs = [
                        x_vmem[pl.ds(base + j * L, L)] for j in range(GS // L)
                    ]
                    av = jnp.abs(xs[0])
                    for j in range(1, GS // L):
                        av = jnp.maximum(av, jnp.abs(xs[j]))
                    amax = jnp.max(av)
                    amaxv = jnp.full((L,), amax, jnp.float32)
                    scale = jnp.where(
                        amaxv == 0.0, INV_MAXQ2, amaxv * INV_MAXQ2
                    )
                    inv = 1.0 / scale
                    for j in range(GS // L):
                        q = xs[j] * inv + zv
                        r = nearest(q)
                        o_vmem[pl.ds(base + j * L, L)] = (r - zv) * scale

        pltpu.emit_pipeline(
            body,
            grid=(n // BLOCK,),
            in_specs=[pl.BlockSpec((BLOCK,), lambda i: (i,))],
            out_specs=[pl.BlockSpec((BLOCK,), lambda i: (i,))],
            core_axis_name=("c", "s"),
            dimension_semantics=(pltpu.PARALLEL,),
        )(x_hbm, o_hbm)

    return run(x1, lookup_values).reshape(shape)


# speedup vs baseline: 1.1666x; 1.1666x over previous
import dataclasses
import functools

import jax
import jax.numpy as jnp
from jax import lax
from jax.experimental import pallas as pl
from jax.experimental.pallas import tpu as pltpu
from jax.experimental.pallas import tpu_sc as plsc

GS = 128
NLEV = 16
L = 16
BLOCK = 16384
GPI = 2
INV_MAXQ2 = 2.0 / 15.0


def kernel(x, lookup_values):
    shape = x.shape
    n = x.size
    x1 = x.reshape(n)
    mesh = plsc.VectorSubcoreMesh(core_axis_name="c", subcore_axis_name="s")
    cp = pltpu.CompilerParams()
    if "needs_layout_passes" in pltpu.CompilerParams.__dataclass_fields__:
        cp = dataclasses.replace(cp, needs_layout_passes=False)

    @functools.partial(
        pl.kernel,
        mesh=mesh,
        out_type=jax.ShapeDtypeStruct((n,), jnp.float32),
        scratch_types=[pltpu.VMEM((NLEV,), jnp.float32)],
        compiler_params=cp,
    )
    def run(x_hbm, lut_hbm, o_hbm, lut_vmem):
        pltpu.sync_copy(lut_hbm, lut_vmem)
        lutv = lut_vmem[...]

        def take(v, idx):
            return v.at[idx].get(mode="promise_in_bounds")

        lane = lax.iota(jnp.int32, L)
        shifted = take(lutv, jnp.minimum(lane + 1, NLEV - 1))
        midsv = (lutv + shifted) * 0.5

        i7 = jnp.full((L,), 7, jnp.int32)
        s8 = jnp.full((L,), 8, jnp.int32)
        s4 = jnp.full((L,), 4, jnp.int32)
        s2 = jnp.full((L,), 2, jnp.int32)
        s1 = jnp.full((L,), 1, jnp.int32)
        s0 = jnp.zeros((L,), jnp.int32)

        def nearest(q):
            lo = jnp.where(q > take(midsv, i7), s8, s0)
            lo = lo + jnp.where(q > take(midsv, lo + 3), s4, s0)
            lo = lo + jnp.where(q > take(midsv, lo + 1), s2, s0)
            lo = lo + jnp.where(q > take(midsv, lo), s1, s0)
            return take(lutv, lo)

        zv = nearest(jnp.full((L,), 8.0, jnp.float32))

        def body(x_vmem, o_vmem):
            @pl.loop(0, BLOCK // GS, step=GPI)
            def _(g0):
                for gg in range(GPI):
                    base = (g0 + gg) * GS
                    xs = [
                        x_vmem[pl.ds(base + j * L, L)] for j in range(GS // L)
                    ]
                    av = jnp.abs(xs[0])
                    for j in range(1, GS // L):
                        av = jnp.maximum(av, jnp.abs(xs[j]))
                    amax = jnp.max(av)
                    amaxv = jnp.full((L,), amax, jnp.float32)
                    scale = jnp.where(
                        amaxv == 0.0, INV_MAXQ2, amaxv * INV_MAXQ2
                    )
                    inv = 1.0 / scale
                    for j in range(GS // L):
                        q = xs[j] * inv + zv
                        r = nearest(q)
                        o_vmem[pl.ds(base + j * L, L)] = (r - zv) * scale

        pltpu.emit_pipeline(
            body,
            grid=(n // BLOCK,),
            in_specs=[pl.BlockSpec((BLOCK,), lambda i: (i,))],
            out_specs=[pl.BlockSpec((BLOCK,), lambda i: (i,))],
            core_axis_name=("c", "s"),
            dimension_semantics=(pltpu.PARALLEL,),
        )(x_hbm, o_hbm)

    return run(x1, lookup_values).reshape(shape)


# speedup vs baseline: 1.3015x; 1.1156x over previous
import dataclasses
import functools

import jax
import jax.numpy as jnp
from jax import lax
from jax.experimental import pallas as pl
from jax.experimental.pallas import tpu as pltpu
from jax.experimental.pallas import tpu_sc as plsc

GS = 128
NLEV = 16
L = 16
BLOCK = 16384
GPI = 2
INV_MAXQ2 = 2.0 / 15.0


def kernel(x, lookup_values):
    shape = x.shape
    n = x.size
    x1 = x.reshape(n)
    mesh = plsc.VectorSubcoreMesh(core_axis_name="c", subcore_axis_name="s")
    cp = pltpu.CompilerParams()
    if "needs_layout_passes" in pltpu.CompilerParams.__dataclass_fields__:
        cp = dataclasses.replace(cp, needs_layout_passes=False)

    @functools.partial(
        pl.kernel,
        mesh=mesh,
        out_type=jax.ShapeDtypeStruct((n,), jnp.float32),
        scratch_types=[pltpu.VMEM((NLEV,), jnp.float32)],
        compiler_params=cp,
    )
    def run(x_hbm, lut_hbm, o_hbm, lut_vmem):
        pltpu.sync_copy(lut_hbm, lut_vmem)
        lutv = lut_vmem[...]

        def take(v, idx):
            return v.at[idx].get(mode="promise_in_bounds")

        lane = lax.iota(jnp.int32, L)
        shifted = take(lutv, jnp.minimum(lane + 1, NLEV - 1))
        midsv = (lutv + shifted) * 0.5

        i7 = jnp.full((L,), 7, jnp.int32)
        s8 = jnp.full((L,), 8, jnp.int32)
        s4 = jnp.full((L,), 4, jnp.int32)
        s2 = jnp.full((L,), 2, jnp.int32)
        s1 = jnp.full((L,), 1, jnp.int32)
        s0 = jnp.zeros((L,), jnp.int32)

        def search(q, msv, valv):
            lo = jnp.where(q > take(msv, i7), s8, s0)
            lo = lo + jnp.where(q > take(msv, lo + 3), s4, s0)
            lo = lo + jnp.where(q > take(msv, lo + 1), s2, s0)
            lo = lo + jnp.where(q > take(msv, lo), s1, s0)
            return take(valv, lo)

        zv = search(jnp.full((L,), 8.0, jnp.float32), midsv, lutv)
        mz = midsv - zv
        oz = lutv - zv

        def body(x_vmem, o_vmem):
            @pl.loop(0, BLOCK // GS, step=GPI)
            def _(g0):
                for gg in range(GPI):
                    base = (g0 + gg) * GS
                    xs = [
                        x_vmem[pl.ds(base + j * L, L)] for j in range(GS // L)
                    ]
                    av = jnp.abs(xs[0])
                    for j in range(1, GS // L):
                        av = jnp.maximum(av, jnp.abs(xs[j]))
                    amax = jnp.max(av)
                    amaxv = jnp.full((L,), amax, jnp.float32)
                    scale = jnp.where(
                        amaxv == 0.0, INV_MAXQ2, amaxv * INV_MAXQ2
                    )
                    msv = mz * scale
                    outv = oz * scale
                    for j in range(GS // L):
                        o_vmem[pl.ds(base + j * L, L)] = search(
                            xs[j], msv, outv
                        )

        pltpu.emit_pipeline(
            body,
            grid=(n // BLOCK,),
            in_specs=[pl.BlockSpec((BLOCK,), lambda i: (i,))],
            out_specs=[pl.BlockSpec((BLOCK,), lambda i: (i,))],
            core_axis_name=("c", "s"),
            dimension_semantics=(pltpu.PARALLEL,),
        )(x_hbm, o_hbm)

    return run(x1, lookup_values).reshape(shape)
